# baseline (speedup 1.0000x reference)
import functools
import jax
import jax.numpy as jnp
from jax import lax
from jax.experimental import pallas as pl
from jax.experimental.pallas import tpu as pltpu
from jax.experimental.pallas import tpu_sc as plsc

_D = 4096
_N_EXP = 64
_TOP_K = 8
_BLK = 1024
_NW = 32
_TPW = 1024


def _gates_block(h_ref, w_ref, gates_ref):
    logits = lax.dot_general(
        h_ref[...], w_ref[...], (((1,), (1,)), ((), ())),
        preferred_element_type=jnp.float32)
    m = jnp.max(logits, axis=-1, keepdims=True)
    e = jnp.exp(logits - m)
    s = jnp.sum(e, axis=-1, keepdims=True)
    gates_ref[...] = e / s


def _tc_gates(hidden_states, W_gate):
    n_tok = hidden_states.shape[0]
    return pl.pallas_call(
        _gates_block,
        grid=(n_tok // _BLK,),
        in_specs=[
            pl.BlockSpec((_BLK, _D), lambda i: (i, 0)),
            pl.BlockSpec((_N_EXP, _D), lambda i: (0, 0)),
        ],
        out_specs=pl.BlockSpec((_BLK, _N_EXP), lambda i: (i, 0)),
        out_shape=jax.ShapeDtypeStruct((n_tok, _N_EXP), jnp.float32),
    )(hidden_states, W_gate)


def _sc_topk(gates_flat, n_tok):
    mesh = plsc.VectorSubcoreMesh(core_axis_name="c", subcore_axis_name="s")

    @functools.partial(
        pl.kernel, mesh=mesh,
        out_type=[
            jax.ShapeDtypeStruct((n_tok * 16,), jnp.float32),
            jax.ShapeDtypeStruct((n_tok * 16,), jnp.int32),
        ],
        scratch_types=[
            pltpu.VMEM((_TPW * _N_EXP,), jnp.float32),
            pltpu.VMEM((_TPW * 16,), jnp.float32),
            pltpu.VMEM((_TPW * 16,), jnp.int32),
        ],
    )
    def k(gates_hbm, gv_hbm, gi_hbm, g_v, v_v, i_v):
        wid = lax.axis_index("s") * 2 + lax.axis_index("c")
        base = wid * _TPW
        pltpu.sync_copy(gates_hbm.at[pl.ds(base * _N_EXP, _TPW * _N_EXP)],
                        g_v)

        col = lax.iota(jnp.int32, 16)
        inv = [63 - (col + 16 * q) for q in range(4)]
        dnums = lax.GatherDimensionNumbers(
            offset_dims=(), collapsed_slice_dims=(0,), start_index_map=(0,))

        def perm(x, idx16):
            return lax.gather(x, idx16[:, None], dnums, (1,),
                              mode=lax.GatherScatterMode.PROMISE_IN_BOUNDS)

        def body(t, carry):
            keys = []
            for q in range(4):
                g = g_v[pl.ds(t * _N_EXP + 16 * q, 16)]
                gb = lax.bitcast_convert_type(g, jnp.int32)
                keys.append(lax.bitcast_convert_type(
                    (gb & ~0x3F) | inv[q], jnp.float32))
            acc = jnp.zeros((16,), jnp.float32)
            for i in range(_TOP_K):
                m = jnp.maximum(jnp.maximum(keys[0], keys[1]),
                                jnp.maximum(keys[2], keys[3]))
                for sh in (1, 2, 4, 8):
                    m = jnp.maximum(m, perm(m, col ^ sh))
                acc = jnp.where(col == i, m, acc)
                keys = [jnp.where(kq == m, -1.0, kq) for kq in keys]
            ab = lax.bitcast_convert_type(acc, jnp.int32)
            v_v[pl.ds(t * 16, 16)] = lax.bitcast_convert_type(
                ab & ~0x3F, jnp.float32)
            i_v[pl.ds(t * 16, 16)] = 63 - (ab & 0x3F)
            return carry

        lax.fori_loop(0, _TPW, body, 0)
        pltpu.sync_copy(v_v, gv_hbm.at[pl.ds(base * 16, _TPW * 16)])
        pltpu.sync_copy(i_v, gi_hbm.at[pl.ds(base * 16, _TPW * 16)])

    return k(gates_flat)


def kernel(hidden_states, W_gate):
    n_tok = hidden_states.shape[0]
    gates = _tc_gates(hidden_states, W_gate)
    gv_pad, gi_pad = _sc_topk(gates.reshape(-1), n_tok)
    gv = gv_pad.reshape(n_tok, 16)[:, :_TOP_K]
    gi = gi_pad.reshape(n_tok, 16)[:, :_TOP_K]
    return gv, gi, gates


# speedup vs baseline: 1.5561x; 1.5561x over previous
import jax
import jax.numpy as jnp
from jax import lax
from jax.experimental import pallas as pl
from jax.experimental.pallas import tpu as pltpu

_D = 4096
_N_EXP = 64
_TOP_K = 8
_BLK = 1024


def _router_block(h_ref, w_ref, gv_ref, gi_ref, gates_ref):
    h = h_ref[...]
    w = w_ref[...]
    logits = lax.dot_general(
        h, w, (((1,), (1,)), ((), ())),
        preferred_element_type=jnp.float32)
    m = jnp.max(logits, axis=-1, keepdims=True)
    e = jnp.exp(logits - m)
    s = jnp.sum(e, axis=-1, keepdims=True)
    gates = e / s
    gates_ref[...] = gates

    col = lax.broadcasted_iota(jnp.int32, (_BLK, _N_EXP), 1)
    gbits = lax.bitcast_convert_type(gates, jnp.int32)
    key = lax.bitcast_convert_type((gbits & ~0x3F) | (63 - col), jnp.float32)
    mxs = []
    for _ in range(_TOP_K):
        mx = jnp.max(key, axis=-1, keepdims=True)
        key = jnp.where(key == mx, -1.0, key)
        mxs.append(mx)
    top = lax.bitcast_convert_type(jnp.concatenate(mxs, axis=1), jnp.int32)
    gv_ref[...] = lax.bitcast_convert_type(top & ~0x3F, jnp.float32)
    gi_ref[...] = 63 - (top & 0x3F)


def kernel(hidden_states, W_gate):
    n_tok = hidden_states.shape[0]
    grid = (n_tok // _BLK,)
    gv, gi, gates = pl.pallas_call(
        _router_block,
        grid=grid,
        in_specs=[
            pl.BlockSpec((_BLK, _D), lambda i: (i, 0)),
            pl.BlockSpec((_N_EXP, _D), lambda i: (0, 0)),
        ],
        out_specs=[
            pl.BlockSpec((_BLK, _TOP_K), lambda i: (i, 0)),
            pl.BlockSpec((_BLK, _TOP_K), lambda i: (i, 0)),
            pl.BlockSpec((_BLK, _N_EXP), lambda i: (i, 0)),
        ],
        out_shape=[
            jax.ShapeDtypeStruct((n_tok, _TOP_K), jnp.float32),
            jax.ShapeDtypeStruct((n_tok, _TOP_K), jnp.int32),
            jax.ShapeDtypeStruct((n_tok, _N_EXP), jnp.float32),
        ],
    )(hidden_states, W_gate)
    return gv, gi, gates


# speedup vs baseline: 1.5580x; 1.0012x over previous
import jax
import jax.numpy as jnp
from jax import lax
from jax.experimental import pallas as pl
from jax.experimental.pallas import tpu as pltpu

_D = 4096
_N_EXP = 64
_TOP_K = 8
_BLK = 512


def _half(h, w, gv_ref, gi_ref, gates_ref, lo):
    sl = pl.ds(lo, _BLK)
    logits = lax.dot_general(
        h, w, (((1,), (1,)), ((), ())),
        preferred_element_type=jnp.float32)
    m = jnp.max(logits, axis=-1, keepdims=True)
    e = jnp.exp(logits - m)
    s = jnp.sum(e, axis=-1, keepdims=True)
    gates = e / s
    gates_ref[sl, :] = gates
    col = lax.broadcasted_iota(jnp.int32, (_BLK, _N_EXP), 1)
    gbits = lax.bitcast_convert_type(gates, jnp.int32)
    key = lax.bitcast_convert_type((gbits & ~0x3F) | (63 - col), jnp.float32)
    mxs = []
    for _ in range(_TOP_K):
        mx = jnp.max(key, axis=-1, keepdims=True)
        key = jnp.where(key == mx, -1.0, key)
        mxs.append(mx)
    top = lax.bitcast_convert_type(jnp.concatenate(mxs, axis=1), jnp.int32)
    gv_ref[sl, :] = lax.bitcast_convert_type(top & ~0x3F, jnp.float32)
    gi_ref[sl, :] = 63 - (top & 0x3F)


def _router_block(h0_ref, h1_ref, w_ref, gv_ref, gi_ref, gates_ref):
    w = w_ref[...]
    _half(h0_ref[...], w, gv_ref, gi_ref, gates_ref, 0)
    _half(h1_ref[...], w, gv_ref, gi_ref, gates_ref, _BLK)


def kernel(hidden_states, W_gate):
    n_tok = hidden_states.shape[0]
    grid = (n_tok // (2 * _BLK),)
    gv, gi, gates = pl.pallas_call(
        _router_block,
        grid=grid,
        in_specs=[
            pl.BlockSpec((_BLK, _D), lambda i: (2 * i, 0)),
            pl.BlockSpec((_BLK, _D), lambda i: (2 * i + 1, 0)),
            pl.BlockSpec((_N_EXP, _D), lambda i: (0, 0)),
        ],
        out_specs=[
            pl.BlockSpec((2 * _BLK, _TOP_K), lambda i: (i, 0)),
            pl.BlockSpec((2 * _BLK, _TOP_K), lambda i: (i, 0)),
            pl.BlockSpec((2 * _BLK, _N_EXP), lambda i: (i, 0)),
        ],
        out_shape=[
            jax.ShapeDtypeStruct((n_tok, _TOP_K), jnp.float32),
            jax.ShapeDtypeStruct((n_tok, _TOP_K), jnp.int32),
            jax.ShapeDtypeStruct((n_tok, _N_EXP), jnp.float32),
        ],
    )(hidden_states, hidden_states, W_gate)
    return gv, gi, gates
